# baseline (speedup 1.0000x reference)
import jax
import jax.numpy as jnp
from jax.experimental import pallas as pl
from jax.experimental.pallas import tpu as pltpu

KH = KW = 5
_PADL = 8
_C0 = _PADL - 2
_BT = 32


def _conv1_kernel(x_ref, m1_ref, b1_ref, o_ref, xe_ref, xo_ref):
    Bt = x_ref.shape[0]
    f32 = jnp.float32
    bf16 = jnp.bfloat16

    xe_ref[...] = jnp.zeros_like(xe_ref)
    xo_ref[...] = jnp.zeros_like(xo_ref)
    xe_ref[:, :, pl.ds(1, 16), :] = x_ref[:, :, pl.ds(0, 16, 2), :].astype(bf16)
    xo_ref[:, :, pl.ds(1, 16), :] = x_ref[:, :, pl.ds(1, 16, 2), :].astype(bf16)
    xe = jnp.concatenate([xe_ref[:, c] for c in range(3)], axis=-1)
    xo = jnp.concatenate([xo_ref[:, c] for c in range(3)], axis=-1)

    def tap(src, off):
        return src[:, off:off + 16].reshape(Bt * 16, 96)

    acc_e = jnp.zeros((Bt * 16, 1024), f32)
    acc_o = jnp.zeros((Bt * 16, 1024), f32)
    for ky in range(KH):
        w_ky = m1_ref[pl.ds(96 * ky, 96), :]
        if ky % 2 == 0:
            lhs_e, lhs_o = tap(xe, ky // 2), tap(xo, ky // 2)
        else:
            lhs_e, lhs_o = tap(xo, (ky - 1) // 2), tap(xe, (ky + 1) // 2)
        acc_e = acc_e + jnp.dot(lhs_e, w_ky, preferred_element_type=f32)
        acc_o = acc_o + jnp.dot(lhs_o, w_ky, preferred_element_type=f32)

    yh = jnp.maximum(acc_e, acc_o)
    p = jnp.maximum(yh[:, :512], yh[:, 512:]) + b1_ref[...]
    z = jnp.zeros((Bt * 16, 64), jnp.float32)
    full = jnp.concatenate([z, p, z], axis=-1).astype(bf16)
    o_ref[:, pl.ds(0, 2), :] = jnp.zeros((Bt, 2, 640), bf16)
    o_ref[:, pl.ds(18, 2), :] = jnp.zeros((Bt, 2, 640), bf16)
    o_ref[:, pl.ds(2, 16), :] = full.reshape(Bt, 16, 640)


def _conv1(x, m1, b1m):
    B = x.shape[0]
    Bt = max(t for t in (1, 2, 4, 8, 16, 32) if t <= B and B % t == 0 and t <= _BT)
    const2 = lambda i: (0, 0)
    return pl.pallas_call(
        _conv1_kernel,
        out_shape=jax.ShapeDtypeStruct((B, 20, 640), jnp.bfloat16),
        grid=(B // Bt,),
        in_specs=[
            pl.BlockSpec((Bt, 3, 32, 32), lambda i: (i, 0, 0, 0)),
            pl.BlockSpec((KH * 96, 1024), const2),
            pl.BlockSpec((1, 512), const2),
        ],
        out_specs=pl.BlockSpec((Bt, 20, 640), lambda i: (i, 0, 0)),
        scratch_shapes=[
            pltpu.VMEM((Bt, 3, 18, 32), jnp.bfloat16),
            pltpu.VMEM((Bt, 3, 18, 32), jnp.bfloat16),
        ],
        compiler_params=pltpu.CompilerParams(
            dimension_semantics=("parallel",),
            vmem_limit_bytes=64 * 1024 * 1024,
        ),
    )(x, m1, b1m)


def _conv23_kernel(xw2_ref, w2_ref, b2_ref, w3_ref, b3_ref,
                   o_ref, pool2_ref, xp3_ref, pool3_ref):
    Bt = xw2_ref.shape[0]
    f32 = jnp.float32
    bf16 = jnp.bfloat16

    xp3_ref[...] = jnp.zeros_like(xp3_ref)

    def conv_ky(xw, H, W, K5, w_ref, b_ref):
        acc = b_ref[...].astype(f32)
        for ky in range(KH):
            m = xw[:, ky:ky + H].reshape(Bt * H * W, K5)
            acc = acc + jnp.dot(m, w_ref[pl.ds(ky * K5, K5), :],
                                preferred_element_type=f32)
        return acc

    def pool2x2(y2d, H, W, C, pool_ref):
        Ho, Wo = H // 2, W // 2
        y = y2d.reshape(Bt * Ho, 2, W, C)
        pool_ref[...] = jnp.maximum(y[:, 0], y[:, 1])
        return jnp.maximum(pool_ref[:, pl.ds(0, Wo, 2), :],
                           pool_ref[:, pl.ds(1, Wo, 2), :]).astype(bf16)

    def wfold(xp_ref, W):
        return jnp.concatenate(
            [xp_ref[:, :, pl.ds(_C0 + kx, W), :] for kx in range(KW)], axis=-1)

    y2 = conv_ky(xw2_ref[...], 16, 16, 160, w2_ref, b2_ref)
    p2 = pool2x2(y2, 16, 16, 32, pool2_ref)
    xp3_ref[:, pl.ds(2, 8), pl.ds(_PADL, 8), :] = p2.reshape(Bt, 8, 8, 32)

    y3 = conv_ky(wfold(xp3_ref, 8), 8, 8, 160, w3_ref, b3_ref)
    o_ref[...] = pool2x2(y3, 8, 8, 64, pool3_ref)


def _conv23(xw2, w2, b2, w3, b3):
    B = xw2.shape[0]
    Bt = max(t for t in (1, 2, 4, 8, 16, 32) if t <= B and B % t == 0 and t <= _BT)
    bf16 = jnp.bfloat16
    f32 = jnp.float32
    const2 = lambda i: (0, 0)
    return pl.pallas_call(
        _conv23_kernel,
        out_shape=jax.ShapeDtypeStruct((B * 4, 4, 64), bf16),
        grid=(B // Bt,),
        in_specs=[
            pl.BlockSpec((Bt, 20, 16, 160), lambda i: (i, 0, 0, 0)),
            pl.BlockSpec((800, 32), const2),
            pl.BlockSpec((1, 32), const2),
            pl.BlockSpec((800, 64), const2),
            pl.BlockSpec((1, 64), const2),
        ],
        out_specs=pl.BlockSpec((Bt * 4, 4, 64), lambda i: (i, 0, 0)),
        scratch_shapes=[
            pltpu.VMEM((Bt * 8, 16, 32), f32),
            pltpu.VMEM((Bt, 12, _PADL + 8 + 2, 32), bf16),
            pltpu.VMEM((Bt * 4, 8, 64), f32),
        ],
        compiler_params=pltpu.CompilerParams(
            dimension_semantics=("parallel",),
            vmem_limit_bytes=64 * 1024 * 1024,
        ),
    )(xw2, w2, b2, w3, b3)


def _dense_head_kernel(x_ref, w1_ref, b1_ref, w2_ref, b2_ref, o_ref):
    h = jnp.dot(x_ref[...], w1_ref[...],
                preferred_element_type=jnp.float32) + b1_ref[...]
    o_ref[...] = jnp.dot(h.astype(jnp.bfloat16), w2_ref[...],
                         preferred_element_type=jnp.float32) + b2_ref[...]


def _dense_head(x, w1, b1, w2, b2):
    B, K = x.shape
    N1, N2 = w1.shape[1], w2.shape[1]
    bm = B if B <= 256 else max(t for t in range(8, 257, 8) if B % t == 0)
    const2 = lambda i: (0, 0)
    return pl.pallas_call(
        _dense_head_kernel,
        out_shape=jax.ShapeDtypeStruct((B, N2), jnp.float32),
        grid=(B // bm,),
        in_specs=[
            pl.BlockSpec((bm, K), lambda i: (i, 0)),
            pl.BlockSpec((K, N1), const2),
            pl.BlockSpec((1, N1), const2),
            pl.BlockSpec((N1, N2), const2),
            pl.BlockSpec((1, N2), const2),
        ],
        out_specs=pl.BlockSpec((bm, N2), lambda i: (i, 0)),
        compiler_params=pltpu.CompilerParams(dimension_semantics=("parallel",)),
    )(x, w1, b1, w2, b2)


def kernel(x, w1i, b1, w2i, b2, w3i, b3, fc1_w_nhwc, fc1_b, fc2_w_pad, fc2_b_pad):
    B = x.shape[0]
    bf16 = jnp.bfloat16
    w1h = w1i.reshape(KH, KW, 3, 32)
    eye = jnp.stack([jnp.eye(32, 32, 2 - kx, dtype=w1i.dtype)
                     for kx in range(KW)])
    m1 = jnp.einsum('xiw,yxco->yciwo', eye, w1h).reshape(KH * 96, 32, 32)
    worder = list(range(0, 32, 2)) + list(range(1, 32, 2))
    m1 = m1[:, worder, :].reshape(KH * 96, 1024).astype(bf16)
    b1m = jnp.tile(b1, 16).reshape(1, 512)

    p1p = _conv1(x, m1, b1m).reshape(B, 20, 20, 32)
    xw2 = jnp.concatenate(
        [p1p[:, :, kx:kx + 16, :] for kx in range(KW)], axis=-1)
    feats = _conv23(xw2,
                    w2i.astype(bf16), b2.reshape(1, 32),
                    w3i.astype(bf16), b3.reshape(1, 64))
    feats = feats.reshape(B, 1024)

    logits = _dense_head(feats, fc1_w_nhwc.astype(bf16), fc1_b.reshape(1, 64),
                         fc2_w_pad.astype(bf16), fc2_b_pad.reshape(1, 128))
    return logits[:, :10]


# speedup vs baseline: 2.3250x; 2.3250x over previous
import jax
import jax.numpy as jnp
from jax.experimental import pallas as pl
from jax.experimental.pallas import tpu as pltpu

KH = KW = 5
_PADL = 8
_C0 = _PADL - 2
_BT = 64


def _conv1_kernel(x_ref, m1_ref, b1_ref, o_ref, xe_ref, xo_ref):
    Bt = x_ref.shape[0]
    f32 = jnp.float32
    bf16 = jnp.bfloat16

    xe_ref[...] = jnp.zeros_like(xe_ref)
    xo_ref[...] = jnp.zeros_like(xo_ref)
    xe_ref[:, :, pl.ds(1, 16), :] = x_ref[:, :, pl.ds(0, 16, 2), :].astype(bf16)
    xo_ref[:, :, pl.ds(1, 16), :] = x_ref[:, :, pl.ds(1, 16, 2), :].astype(bf16)
    xe = jnp.concatenate([xe_ref[:, c] for c in range(3)], axis=-1)
    xo = jnp.concatenate([xo_ref[:, c] for c in range(3)], axis=-1)

    def tap(src, off):
        return src[:, off:off + 16].reshape(Bt * 16, 96)

    acc_e = jnp.zeros((Bt * 16, 1024), f32)
    acc_o = jnp.zeros((Bt * 16, 1024), f32)
    for ky in range(KH):
        w_ky = m1_ref[pl.ds(96 * ky, 96), :]
        if ky % 2 == 0:
            lhs_e, lhs_o = tap(xe, ky // 2), tap(xo, ky // 2)
        else:
            lhs_e, lhs_o = tap(xo, (ky - 1) // 2), tap(xe, (ky + 1) // 2)
        acc_e = acc_e + jnp.dot(lhs_e, w_ky, preferred_element_type=f32)
        acc_o = acc_o + jnp.dot(lhs_o, w_ky, preferred_element_type=f32)

    yh = jnp.maximum(acc_e, acc_o)
    p = jnp.maximum(yh[:, :512], yh[:, 512:]) + b1_ref[...]
    o_ref[...] = p.astype(bf16).reshape(Bt, 16, 512)


def _conv1(x, m1, b1m):
    B = x.shape[0]
    Bt = max(t for t in (1, 2, 4, 8, 16, 32, 64) if t <= B and B % t == 0 and t <= _BT)
    const2 = lambda i: (0, 0)
    return pl.pallas_call(
        _conv1_kernel,
        out_shape=jax.ShapeDtypeStruct((B, 16, 512), jnp.bfloat16),
        grid=(B // Bt,),
        in_specs=[
            pl.BlockSpec((Bt, 3, 32, 32), lambda i: (i, 0, 0, 0)),
            pl.BlockSpec((KH * 96, 1024), const2),
            pl.BlockSpec((1, 512), const2),
        ],
        out_specs=pl.BlockSpec((Bt, 16, 512), lambda i: (i, 0, 0)),
        scratch_shapes=[
            pltpu.VMEM((Bt, 3, 18, 32), jnp.bfloat16),
            pltpu.VMEM((Bt, 3, 18, 32), jnp.bfloat16),
        ],
        compiler_params=pltpu.CompilerParams(
            dimension_semantics=("parallel",),
            vmem_limit_bytes=64 * 1024 * 1024,
        ),
    )(x, m1, b1m)


def _conv23_kernel(p1_ref, w2_ref, b2_ref, w3_ref, b3_ref,
                   o_ref, xp2_ref, pool2_ref, xp3_ref, pool3_ref):
    Bt = p1_ref.shape[0]
    f32 = jnp.float32
    bf16 = jnp.bfloat16

    def zero_halo(ref, Hp, Wp):
        z = lambda s: jnp.zeros(s, ref.dtype)
        ref[:, pl.ds(0, 2), :, :] = z((Bt, 2, Wp, 32))
        ref[:, pl.ds(Hp - 2, 2), :, :] = z((Bt, 2, Wp, 32))
        ref[:, pl.ds(2, Hp - 4), pl.ds(0, 2), :] = z((Bt, Hp - 4, 2, 32))
        ref[:, pl.ds(2, Hp - 4), pl.ds(Wp - 2, 2), :] = z((Bt, Hp - 4, 2, 32))
    zero_halo(xp2_ref, 20, 20)
    zero_halo(xp3_ref, 12, 12)

    def conv_ky(xw, H, W, K5, w_ref, b_ref):
        acc = b_ref[...].astype(f32)
        for ky in range(KH):
            m = xw[:, ky:ky + H].reshape(Bt * H * W, K5)
            acc = acc + jnp.dot(m, w_ref[pl.ds(ky * K5, K5), :],
                                preferred_element_type=f32)
        return acc

    def pool2x2(y2d, H, W, C, pool_ref):
        Ho, Wo = H // 2, W // 2
        y = y2d.reshape(Bt * Ho, 2, W, C)
        pool_ref[...] = jnp.maximum(y[:, 0], y[:, 1])
        return jnp.maximum(pool_ref[:, pl.ds(0, Wo, 2), :],
                           pool_ref[:, pl.ds(1, Wo, 2), :]).astype(bf16)

    def wfold(xp_ref, W):
        return jnp.concatenate(
            [xp_ref[:, :, pl.ds(kx, W), :] for kx in range(KW)], axis=-1)

    xp2_ref[:, pl.ds(2, 16), pl.ds(2, 16), :] = p1_ref[...]

    y2 = conv_ky(wfold(xp2_ref, 16), 16, 16, 160, w2_ref, b2_ref)
    p2 = pool2x2(y2, 16, 16, 32, pool2_ref)
    xp3_ref[:, pl.ds(2, 8), pl.ds(2, 8), :] = p2.reshape(Bt, 8, 8, 32)

    y3 = conv_ky(wfold(xp3_ref, 8), 8, 8, 160, w3_ref, b3_ref)
    o_ref[...] = pool2x2(y3, 8, 8, 64, pool3_ref)


def _conv23(p1, w2, b2, w3, b3):
    B = p1.shape[0]
    Bt = max(t for t in (1, 2, 4, 8, 16, 32, 64) if t <= B and B % t == 0 and t <= _BT)
    bf16 = jnp.bfloat16
    f32 = jnp.float32
    const2 = lambda i: (0, 0)
    return pl.pallas_call(
        _conv23_kernel,
        out_shape=jax.ShapeDtypeStruct((B * 4, 4, 64), bf16),
        grid=(B // Bt,),
        in_specs=[
            pl.BlockSpec((Bt, 16, 16, 32), lambda i: (i, 0, 0, 0)),
            pl.BlockSpec((800, 32), const2),
            pl.BlockSpec((1, 32), const2),
            pl.BlockSpec((800, 64), const2),
            pl.BlockSpec((1, 64), const2),
        ],
        out_specs=pl.BlockSpec((Bt * 4, 4, 64), lambda i: (i, 0, 0)),
        scratch_shapes=[
            pltpu.VMEM((Bt, 20, 20, 32), bf16),
            pltpu.VMEM((Bt * 8, 16, 32), f32),
            pltpu.VMEM((Bt, 12, 12, 32), bf16),
            pltpu.VMEM((Bt * 4, 8, 64), f32),
        ],
        compiler_params=pltpu.CompilerParams(
            dimension_semantics=("parallel",),
            vmem_limit_bytes=64 * 1024 * 1024,
        ),
    )(p1, w2, b2, w3, b3)


def _dense_head_kernel(x_ref, w1_ref, b1_ref, w2_ref, b2_ref, o_ref):
    h = jnp.dot(x_ref[...], w1_ref[...],
                preferred_element_type=jnp.float32) + b1_ref[...]
    o_ref[...] = jnp.dot(h.astype(jnp.bfloat16), w2_ref[...],
                         preferred_element_type=jnp.float32) + b2_ref[...]


def _dense_head(x, w1, b1, w2, b2):
    B, K = x.shape
    N1, N2 = w1.shape[1], w2.shape[1]
    bm = B if B <= 256 else max(t for t in range(8, 257, 8) if B % t == 0)
    const2 = lambda i: (0, 0)
    return pl.pallas_call(
        _dense_head_kernel,
        out_shape=jax.ShapeDtypeStruct((B, N2), jnp.float32),
        grid=(B // bm,),
        in_specs=[
            pl.BlockSpec((bm, K), lambda i: (i, 0)),
            pl.BlockSpec((K, N1), const2),
            pl.BlockSpec((1, N1), const2),
            pl.BlockSpec((N1, N2), const2),
            pl.BlockSpec((1, N2), const2),
        ],
        out_specs=pl.BlockSpec((bm, N2), lambda i: (i, 0)),
        compiler_params=pltpu.CompilerParams(dimension_semantics=("parallel",)),
    )(x, w1, b1, w2, b2)


def kernel(x, w1i, b1, w2i, b2, w3i, b3, fc1_w_nhwc, fc1_b, fc2_w_pad, fc2_b_pad):
    B = x.shape[0]
    bf16 = jnp.bfloat16
    w1h = w1i.reshape(KH, KW, 3, 32)
    eye = jnp.stack([jnp.eye(32, 32, 2 - kx, dtype=w1i.dtype)
                     for kx in range(KW)])
    m1 = jnp.einsum('xiw,yxco->yciwo', eye, w1h).reshape(KH * 96, 32, 32)
    worder = list(range(0, 32, 2)) + list(range(1, 32, 2))
    m1 = m1[:, worder, :].reshape(KH * 96, 1024).astype(bf16)
    b1m = jnp.tile(b1, 16).reshape(1, 512)

    p1 = _conv1(x, m1, b1m).reshape(B, 16, 16, 32)
    feats = _conv23(p1,
                    w2i.astype(bf16), b2.reshape(1, 32),
                    w3i.astype(bf16), b3.reshape(1, 64))
    feats = feats.reshape(B, 1024)

    logits = _dense_head(feats, fc1_w_nhwc.astype(bf16), fc1_b.reshape(1, 64),
                         fc2_w_pad.astype(bf16), fc2_b_pad.reshape(1, 128))
    return logits[:, :10]


# speedup vs baseline: 2.3479x; 1.0099x over previous
import jax
import jax.numpy as jnp
from jax.experimental import pallas as pl
from jax.experimental.pallas import tpu as pltpu

KH = KW = 5
_PADL = 8
_C0 = _PADL - 2
_BT = 32


def _conv1_kernel(x_ref, m1_ref, b1_ref, o_ref, xe_ref, xo_ref):
    Bt = x_ref.shape[0]
    f32 = jnp.float32
    bf16 = jnp.bfloat16

    xe_ref[...] = jnp.zeros_like(xe_ref)
    xo_ref[...] = jnp.zeros_like(xo_ref)
    xe_ref[:, :, pl.ds(1, 16), :] = x_ref[:, :, pl.ds(0, 16, 2), :].astype(bf16)
    xo_ref[:, :, pl.ds(1, 16), :] = x_ref[:, :, pl.ds(1, 16, 2), :].astype(bf16)
    xe = jnp.concatenate([xe_ref[:, c] for c in range(3)], axis=-1)
    xo = jnp.concatenate([xo_ref[:, c] for c in range(3)], axis=-1)

    def tap(src, off):
        return src[:, off:off + 16].reshape(Bt * 16, 96)

    acc_e = jnp.zeros((Bt * 16, 1024), f32)
    acc_o = jnp.zeros((Bt * 16, 1024), f32)
    for ky in range(KH):
        w_ky = m1_ref[pl.ds(96 * ky, 96), :]
        if ky % 2 == 0:
            lhs_e, lhs_o = tap(xe, ky // 2), tap(xo, ky // 2)
        else:
            lhs_e, lhs_o = tap(xo, (ky - 1) // 2), tap(xe, (ky + 1) // 2)
        acc_e = acc_e + jnp.dot(lhs_e, w_ky, preferred_element_type=f32)
        acc_o = acc_o + jnp.dot(lhs_o, w_ky, preferred_element_type=f32)

    yh = jnp.maximum(acc_e, acc_o)
    p = jnp.maximum(yh[:, :512], yh[:, 512:]) + b1_ref[...]
    o_ref[...] = p.astype(bf16).reshape(Bt, 16, 512)


def _conv1(x, m1, b1m):
    B = x.shape[0]
    Bt = max(t for t in (1, 2, 4, 8, 16, 32, 64) if t <= B and B % t == 0 and t <= _BT)
    const2 = lambda i: (0, 0)
    return pl.pallas_call(
        _conv1_kernel,
        out_shape=jax.ShapeDtypeStruct((B, 16, 512), jnp.bfloat16),
        grid=(B // Bt,),
        in_specs=[
            pl.BlockSpec((Bt, 3, 32, 32), lambda i: (i, 0, 0, 0)),
            pl.BlockSpec((KH * 96, 1024), const2),
            pl.BlockSpec((1, 512), const2),
        ],
        out_specs=pl.BlockSpec((Bt, 16, 512), lambda i: (i, 0, 0)),
        scratch_shapes=[
            pltpu.VMEM((Bt, 3, 18, 32), jnp.bfloat16),
            pltpu.VMEM((Bt, 3, 18, 32), jnp.bfloat16),
        ],
        compiler_params=pltpu.CompilerParams(
            dimension_semantics=("parallel",),
            vmem_limit_bytes=64 * 1024 * 1024,
        ),
    )(x, m1, b1m)


def _conv23_kernel(p1_ref, w2_ref, b2_ref, w3_ref, b3_ref,
                   o_ref, xp2_ref, pool2_ref, xp3_ref, pool3_ref):
    Bt = p1_ref.shape[0]
    f32 = jnp.float32
    bf16 = jnp.bfloat16

    def zero_halo(ref, Hp, Wp):
        z = lambda s: jnp.zeros(s, ref.dtype)
        ref[:, pl.ds(0, 2), :, :] = z((Bt, 2, Wp, 32))
        ref[:, pl.ds(Hp - 2, 2), :, :] = z((Bt, 2, Wp, 32))
        ref[:, pl.ds(2, Hp - 4), pl.ds(0, 2), :] = z((Bt, Hp - 4, 2, 32))
        ref[:, pl.ds(2, Hp - 4), pl.ds(Wp - 2, 2), :] = z((Bt, Hp - 4, 2, 32))
    zero_halo(xp2_ref, 20, 20)
    zero_halo(xp3_ref, 12, 12)

    def conv_ky(xw, H, W, K5, w_ref, b_ref):
        acc = b_ref[...].astype(f32)
        for ky in range(KH):
            m = xw[:, ky:ky + H].reshape(Bt * H * W, K5)
            acc = acc + jnp.dot(m, w_ref[pl.ds(ky * K5, K5), :],
                                preferred_element_type=f32)
        return acc

    def pool2x2(y2d, H, W, C, pool_ref):
        Ho, Wo = H // 2, W // 2
        y = y2d.reshape(Bt * Ho, 2, W, C)
        pool_ref[...] = jnp.maximum(y[:, 0], y[:, 1])
        return jnp.maximum(pool_ref[:, pl.ds(0, Wo, 2), :],
                           pool_ref[:, pl.ds(1, Wo, 2), :]).astype(bf16)

    def wfold(xp_ref, W):
        return jnp.concatenate(
            [xp_ref[:, :, pl.ds(kx, W), :] for kx in range(KW)], axis=-1)

    xp2_ref[:, pl.ds(2, 16), pl.ds(2, 16), :] = p1_ref[...]

    y2 = conv_ky(wfold(xp2_ref, 16), 16, 16, 160, w2_ref, b2_ref)
    p2 = pool2x2(y2, 16, 16, 32, pool2_ref)
    xp3_ref[:, pl.ds(2, 8), pl.ds(2, 8), :] = p2.reshape(Bt, 8, 8, 32)

    y3 = conv_ky(wfold(xp3_ref, 8), 8, 8, 160, w3_ref, b3_ref)
    o_ref[...] = pool2x2(y3, 8, 8, 64, pool3_ref)


def _conv23(p1, w2, b2, w3, b3):
    B = p1.shape[0]
    Bt = max(t for t in (1, 2, 4, 8, 16, 32, 64) if t <= B and B % t == 0 and t <= _BT)
    bf16 = jnp.bfloat16
    f32 = jnp.float32
    const2 = lambda i: (0, 0)
    return pl.pallas_call(
        _conv23_kernel,
        out_shape=jax.ShapeDtypeStruct((B * 4, 4, 64), bf16),
        grid=(B // Bt,),
        in_specs=[
            pl.BlockSpec((Bt, 16, 16, 32), lambda i: (i, 0, 0, 0)),
            pl.BlockSpec((800, 32), const2),
            pl.BlockSpec((1, 32), const2),
            pl.BlockSpec((800, 64), const2),
            pl.BlockSpec((1, 64), const2),
        ],
        out_specs=pl.BlockSpec((Bt * 4, 4, 64), lambda i: (i, 0, 0)),
        scratch_shapes=[
            pltpu.VMEM((Bt, 20, 20, 32), bf16),
            pltpu.VMEM((Bt * 8, 16, 32), f32),
            pltpu.VMEM((Bt, 12, 12, 32), bf16),
            pltpu.VMEM((Bt * 4, 8, 64), f32),
        ],
        compiler_params=pltpu.CompilerParams(
            dimension_semantics=("parallel",),
            vmem_limit_bytes=64 * 1024 * 1024,
        ),
    )(p1, w2, b2, w3, b3)


def _dense_head_kernel(x_ref, w1_ref, b1_ref, w2_ref, b2_ref, o_ref):
    h = jnp.dot(x_ref[...], w1_ref[...],
                preferred_element_type=jnp.float32) + b1_ref[...]
    o_ref[...] = jnp.dot(h.astype(jnp.bfloat16), w2_ref[...],
                         preferred_element_type=jnp.float32) + b2_ref[...]


def _dense_head(x, w1, b1, w2, b2):
    B, K = x.shape
    N1, N2 = w1.shape[1], w2.shape[1]
    bm = B if B <= 256 else max(t for t in range(8, 257, 8) if B % t == 0)
    const2 = lambda i: (0, 0)
    return pl.pallas_call(
        _dense_head_kernel,
        out_shape=jax.ShapeDtypeStruct((B, N2), jnp.float32),
        grid=(B // bm,),
        in_specs=[
            pl.BlockSpec((bm, K), lambda i: (i, 0)),
            pl.BlockSpec((K, N1), const2),
            pl.BlockSpec((1, N1), const2),
            pl.BlockSpec((N1, N2), const2),
            pl.BlockSpec((1, N2), const2),
        ],
        out_specs=pl.BlockSpec((bm, N2), lambda i: (i, 0)),
        compiler_params=pltpu.CompilerParams(dimension_semantics=("parallel",)),
    )(x, w1, b1, w2, b2)


def kernel(x, w1i, b1, w2i, b2, w3i, b3, fc1_w_nhwc, fc1_b, fc2_w_pad, fc2_b_pad):
    B = x.shape[0]
    bf16 = jnp.bfloat16
    w1h = w1i.reshape(KH, KW, 3, 32)
    eye = jnp.stack([jnp.eye(32, 32, 2 - kx, dtype=w1i.dtype)
                     for kx in range(KW)])
    m1 = jnp.einsum('xiw,yxco->yciwo', eye, w1h).reshape(KH * 96, 32, 32)
    worder = list(range(0, 32, 2)) + list(range(1, 32, 2))
    m1 = m1[:, worder, :].reshape(KH * 96, 1024).astype(bf16)
    b1m = jnp.tile(b1, 16).reshape(1, 512)

    p1 = _conv1(x, m1, b1m).reshape(B, 16, 16, 32)
    feats = _conv23(p1,
                    w2i.astype(bf16), b2.reshape(1, 32),
                    w3i.astype(bf16), b3.reshape(1, 64))
    feats = feats.reshape(B, 1024)

    logits = _dense_head(feats, fc1_w_nhwc.astype(bf16), fc1_b.reshape(1, 64),
                         fc2_w_pad.astype(bf16), fc2_b_pad.reshape(1, 128))
    return logits[:, :10]


# speedup vs baseline: 2.4173x; 1.0295x over previous
import jax
import jax.numpy as jnp
from jax.experimental import pallas as pl
from jax.experimental.pallas import tpu as pltpu

KH = KW = 5
_PADL = 8
_C0 = _PADL - 2
_BT = 32


def _conv1_kernel(x_ref, m1_ref, b1_ref, o_ref, xe_ref, xo_ref):
    Bt = x_ref.shape[0]
    f32 = jnp.float32
    bf16 = jnp.bfloat16

    xe_ref[...] = jnp.zeros_like(xe_ref)
    xo_ref[...] = jnp.zeros_like(xo_ref)
    xe_ref[:, :, pl.ds(1, 16), :] = x_ref[:, :, pl.ds(0, 16, 2), :].astype(bf16)
    xo_ref[:, :, pl.ds(1, 16), :] = x_ref[:, :, pl.ds(1, 16, 2), :].astype(bf16)
    xe = jnp.concatenate([xe_ref[:, c] for c in range(3)], axis=-1)
    xo = jnp.concatenate([xo_ref[:, c] for c in range(3)], axis=-1)

    def tap(src, off):
        return src[:, off:off + 16].reshape(Bt * 16, 96)

    acc_e = jnp.zeros((Bt * 16, 1024), f32)
    acc_o = jnp.zeros((Bt * 16, 1024), f32)
    for ky in range(KH):
        w_ky = m1_ref[pl.ds(96 * ky, 96), :]
        if ky % 2 == 0:
            lhs_e, lhs_o = tap(xe, ky // 2), tap(xo, ky // 2)
        else:
            lhs_e, lhs_o = tap(xo, (ky - 1) // 2), tap(xe, (ky + 1) // 2)
        acc_e = acc_e + jnp.dot(lhs_e, w_ky, preferred_element_type=f32)
        acc_o = acc_o + jnp.dot(lhs_o, w_ky, preferred_element_type=f32)

    yh = jnp.maximum(acc_e, acc_o)
    p = jnp.maximum(yh[:, :512], yh[:, 512:]) + b1_ref[...]
    o_ref[...] = p.astype(bf16).reshape(Bt, 16, 512)


def _conv1(x, m1, b1m):
    B = x.shape[0]
    Bt = max(t for t in (1, 2, 4, 8, 16, 32) if t <= B and B % t == 0 and t <= _BT)
    const2 = lambda i: (0, 0)
    return pl.pallas_call(
        _conv1_kernel,
        out_shape=jax.ShapeDtypeStruct((B, 16, 512), jnp.bfloat16),
        grid=(B // Bt,),
        in_specs=[
            pl.BlockSpec((Bt, 3, 32, 32), lambda i: (i, 0, 0, 0)),
            pl.BlockSpec((KH * 96, 1024), const2),
            pl.BlockSpec((1, 512), const2),
        ],
        out_specs=pl.BlockSpec((Bt, 16, 512), lambda i: (i, 0, 0)),
        scratch_shapes=[
            pltpu.VMEM((Bt, 3, 18, 32), jnp.bfloat16),
            pltpu.VMEM((Bt, 3, 18, 32), jnp.bfloat16),
        ],
        compiler_params=pltpu.CompilerParams(
            dimension_semantics=("parallel",),
            vmem_limit_bytes=64 * 1024 * 1024,
        ),
    )(x, m1, b1m)


def _conv23_kernel(p1_ref, w2_ref, b2_ref, w3_ref, b3_ref,
                   o_ref, xp2_ref, pool2_ref, xp3_ref, pool3_ref):
    Bt = p1_ref.shape[0]
    f32 = jnp.float32
    bf16 = jnp.bfloat16

    xp2_ref[...] = jnp.zeros_like(xp2_ref)
    xp3_ref[...] = jnp.zeros_like(xp3_ref)

    def conv_ky(xw, H, W, K5, w_ref, b_ref):
        acc = b_ref[...].astype(f32)
        for ky in range(KH):
            m = xw[:, ky:ky + H].reshape(Bt * H * W, K5)
            acc = acc + jnp.dot(m, w_ref[pl.ds(ky * K5, K5), :],
                                preferred_element_type=f32)
        return acc

    def pool2x2(y2d, H, W, C, pool_ref):
        Ho, Wo = H // 2, W // 2
        y = y2d.reshape(Bt * Ho, 2, W, C)
        pool_ref[...] = jnp.maximum(y[:, 0], y[:, 1])
        return jnp.maximum(pool_ref[:, pl.ds(0, Wo, 2), :],
                           pool_ref[:, pl.ds(1, Wo, 2), :]).astype(bf16)

    def wfold(xp_ref, W):
        return jnp.concatenate(
            [xp_ref[:, :, pl.ds(_C0 + kx, W), :] for kx in range(KW)], axis=-1)

    xp2_ref[:, pl.ds(2, 16), pl.ds(_PADL, 16), :] = p1_ref[...]

    y2 = conv_ky(wfold(xp2_ref, 16), 16, 16, 160, w2_ref, b2_ref)
    p2 = pool2x2(y2, 16, 16, 32, pool2_ref)
    xp3_ref[:, pl.ds(2, 8), pl.ds(_PADL, 8), :] = p2.reshape(Bt, 8, 8, 32)

    y3 = conv_ky(wfold(xp3_ref, 8), 8, 8, 160, w3_ref, b3_ref)
    o_ref[...] = pool2x2(y3, 8, 8, 64, pool3_ref)


def _conv23(p1, w2, b2, w3, b3):
    B = p1.shape[0]
    Bt = max(t for t in (1, 2, 4, 8, 16, 32) if t <= B and B % t == 0 and t <= _BT)
    bf16 = jnp.bfloat16
    f32 = jnp.float32
    const2 = lambda i: (0, 0)
    return pl.pallas_call(
        _conv23_kernel,
        out_shape=jax.ShapeDtypeStruct((B * 4, 4, 64), bf16),
        grid=(B // Bt,),
        in_specs=[
            pl.BlockSpec((Bt, 16, 16, 32), lambda i: (i, 0, 0, 0)),
            pl.BlockSpec((800, 32), const2),
            pl.BlockSpec((1, 32), const2),
            pl.BlockSpec((800, 64), const2),
            pl.BlockSpec((1, 64), const2),
        ],
        out_specs=pl.BlockSpec((Bt * 4, 4, 64), lambda i: (i, 0, 0)),
        scratch_shapes=[
            pltpu.VMEM((Bt, 20, _PADL + 16 + 2, 32), bf16),
            pltpu.VMEM((Bt * 8, 16, 32), f32),
            pltpu.VMEM((Bt, 12, _PADL + 8 + 2, 32), bf16),
            pltpu.VMEM((Bt * 4, 8, 64), f32),
        ],
        compiler_params=pltpu.CompilerParams(
            dimension_semantics=("parallel",),
            vmem_limit_bytes=64 * 1024 * 1024,
        ),
    )(p1, w2, b2, w3, b3)


def _dense_head_kernel(x_ref, w1_ref, b1_ref, w2_ref, b2_ref, o_ref):
    h = jnp.dot(x_ref[...], w1_ref[...],
                preferred_element_type=jnp.float32) + b1_ref[...]
    o_ref[...] = jnp.dot(h.astype(jnp.bfloat16), w2_ref[...],
                         preferred_element_type=jnp.float32) + b2_ref[...]


def _dense_head(x, w1, b1, w2, b2):
    B, K = x.shape
    N1, N2 = w1.shape[1], w2.shape[1]
    bm = B if B <= 256 else max(t for t in range(8, 257, 8) if B % t == 0)
    const2 = lambda i: (0, 0)
    return pl.pallas_call(
        _dense_head_kernel,
        out_shape=jax.ShapeDtypeStruct((B, N2), jnp.float32),
        grid=(B // bm,),
        in_specs=[
            pl.BlockSpec((bm, K), lambda i: (i, 0)),
            pl.BlockSpec((K, N1), const2),
            pl.BlockSpec((1, N1), const2),
            pl.BlockSpec((N1, N2), const2),
            pl.BlockSpec((1, N2), const2),
        ],
        out_specs=pl.BlockSpec((bm, N2), lambda i: (i, 0)),
        compiler_params=pltpu.CompilerParams(dimension_semantics=("parallel",)),
    )(x, w1, b1, w2, b2)


def kernel(x, w1i, b1, w2i, b2, w3i, b3, fc1_w_nhwc, fc1_b, fc2_w_pad, fc2_b_pad):
    B = x.shape[0]
    bf16 = jnp.bfloat16
    w1h = w1i.reshape(KH, KW, 3, 32)
    eye = jnp.stack([jnp.eye(32, 32, 2 - kx, dtype=w1i.dtype)
                     for kx in range(KW)])
    m1 = jnp.einsum('xiw,yxco->yciwo', eye, w1h).reshape(KH * 96, 32, 32)
    worder = list(range(0, 32, 2)) + list(range(1, 32, 2))
    m1 = m1[:, worder, :].reshape(KH * 96, 1024).astype(bf16)
    b1m = jnp.tile(b1, 16).reshape(1, 512)

    p1 = _conv1(x, m1, b1m).reshape(B, 16, 16, 32)
    feats = _conv23(p1,
                    w2i.astype(bf16), b2.reshape(1, 32),
                    w3i.astype(bf16), b3.reshape(1, 64))
    feats = feats.reshape(B, 1024)

    logits = _dense_head(feats, fc1_w_nhwc.astype(bf16), fc1_b.reshape(1, 64),
                         fc2_w_pad.astype(bf16), fc2_b_pad.reshape(1, 128))
    return logits[:, :10]
